# Initial kernel scaffold; baseline (speedup 1.0000x reference)
import jax
import jax.numpy as jnp
from jax.experimental import pallas as pl


def kernel(x, edge_index, edge_weight, W, b):
    raise NotImplementedError("write your pallas kernel here")


# speedup vs baseline: 7.7417x; 7.7417x over previous
import functools

import jax
import jax.numpy as jnp
from jax import lax
from jax.experimental import pallas as pl
from jax.experimental.pallas import tpu as pltpu
from jax.experimental.pallas import tpu_sc as plsc

N = 10000
E = 320000
D = 128

NC = 2
NS = 16
NW = NC * NS
NPAD = 10240
SPT = NPAD // NS
EPW = E // NW
SUB = 80
NSUB = 5
G = SUB * NSUB
NSB = EPW // G


def _mesh():
    return plsc.VectorSubcoreMesh(
        core_axis_name="c", subcore_axis_name="s", num_cores=NC, num_subcores=NS
    )


def _sc_degrees(src, dst):
    @functools.partial(
        pl.kernel,
        mesh=_mesh(),
        out_type=(
            jax.ShapeDtypeStruct((NC * NPAD,), jnp.float32),
            jax.ShapeDtypeStruct((NC * NPAD,), jnp.float32),
        ),
        scratch_types=[
            pltpu.VMEM_SHARED((NPAD,), jnp.float32),
            pltpu.VMEM_SHARED((NPAD,), jnp.float32),
            pltpu.VMEM((SUB,), jnp.float32),
            pltpu.VMEM((SUB,), jnp.float32),
            pltpu.VMEM((4 * SUB,), jnp.int32),
            pltpu.VMEM((4 * SUB,), jnp.int32),
            pltpu.VMEM((SPT,), jnp.float32),
            pltpu.SemaphoreType.DMA,
            pltpu.SemaphoreType.DMA,
        ],
    )
    def k(src_h, dst_h, dego_h, degi_h, ho_sh, hi_sh, ones_v, zo_v, sq, dq,
          zv, ld_sem, sc_sem):
        c = lax.axis_index("c")
        s = lax.axis_index("s")
        wid = s * NC + c
        ebase = wid * EPW

        def fill1(i, carry):
            ones_v[pl.ds(i * 16, 16)] = jnp.ones((16,), jnp.float32)
            zo_v[pl.ds(i * 16, 16)] = jnp.zeros((16,), jnp.float32)
            return carry

        lax.fori_loop(0, SUB // 16, fill1, 0)

        def fillq(i, carry):
            sq[pl.ds(i * 16, 16)] = jnp.zeros((16,), jnp.int32)
            dq[pl.ds(i * 16, 16)] = jnp.zeros((16,), jnp.int32)
            return carry

        lax.fori_loop(0, (4 * SUB) // 16, fillq, 0)

        def fill0(i, carry):
            zv[pl.ds(i * 16, 16)] = jnp.zeros((16,), jnp.float32)
            return carry

        lax.fori_loop(0, SPT // 16, fill0, 0)

        pltpu.sync_copy(zv, ho_sh.at[pl.ds(s * SPT, SPT)])
        pltpu.sync_copy(zv, hi_sh.at[pl.ds(s * SPT, SPT)])
        plsc.subcore_barrier()

        def q4(t):
            return lax.rem(t, 4) * SUB

        def issue_loads(t):
            pltpu.async_copy(src_h.at[pl.ds(ebase + t * SUB, SUB)],
                             sq.at[pl.ds(q4(t), SUB)], ld_sem)
            pltpu.async_copy(dst_h.at[pl.ds(ebase + t * SUB, SUB)],
                             dq.at[pl.ds(q4(t), SUB)], ld_sem)

        def wait_loads(t):
            pltpu.make_async_copy(src_h.at[pl.ds(ebase + t * SUB, SUB)],
                                  sq.at[pl.ds(q4(t), SUB)], ld_sem).wait()
            pltpu.make_async_copy(dst_h.at[pl.ds(ebase + t * SUB, SUB)],
                                  dq.at[pl.ds(q4(t), SUB)], ld_sem).wait()

        pltpu.async_copy(zo_v, ho_sh.at[sq.at[pl.ds(2 * SUB, SUB)]],
                         sc_sem, add=True)
        pltpu.async_copy(zo_v, hi_sh.at[dq.at[pl.ds(2 * SUB, SUB)]],
                         sc_sem, add=True)
        pltpu.async_copy(zo_v, ho_sh.at[sq.at[pl.ds(3 * SUB, SUB)]],
                         sc_sem, add=True)
        pltpu.async_copy(zo_v, hi_sh.at[dq.at[pl.ds(3 * SUB, SUB)]],
                         sc_sem, add=True)
        issue_loads(0)
        issue_loads(1)

        NB = EPW // SUB

        def hbody(bi, carry):
            pltpu.make_async_copy(
                ones_v, ho_sh.at[sq.at[pl.ds(q4(bi + 2), SUB)]],
                sc_sem).wait()
            pltpu.make_async_copy(
                ones_v, hi_sh.at[dq.at[pl.ds(q4(bi + 2), SUB)]],
                sc_sem).wait()
            issue_loads(bi + 2)
            wait_loads(bi)
            pltpu.async_copy(ones_v, ho_sh.at[sq.at[pl.ds(q4(bi), SUB)]],
                             sc_sem, add=True)
            pltpu.async_copy(ones_v, hi_sh.at[dq.at[pl.ds(q4(bi), SUB)]],
                             sc_sem, add=True)
            return carry

        lax.fori_loop(0, NB, hbody, 0)

        for t in (NB - 2, NB - 1):
            pltpu.make_async_copy(
                ones_v, ho_sh.at[sq.at[pl.ds(q4(t), SUB)]], sc_sem).wait()
            pltpu.make_async_copy(
                ones_v, hi_sh.at[dq.at[pl.ds(q4(t), SUB)]], sc_sem).wait()
        for t in (NB, NB + 1):
            wait_loads(t)
        plsc.subcore_barrier()

        for sh, outh in ((ho_sh, dego_h), (hi_sh, degi_h)):
            pltpu.sync_copy(sh.at[pl.ds(s * SPT, SPT)], zv)
            pltpu.sync_copy(zv, outh.at[pl.ds(c * NPAD + s * SPT, SPT)])

    return k(src, dst)


def _tc_scales(a, b, cc, d):
    def body(a_ref, b_ref, c_ref, d_ref, so_ref, si_ref):
        so_ref[...] = lax.rsqrt(jnp.maximum(a_ref[...] + b_ref[...], 1.0))
        si_ref[...] = lax.rsqrt(jnp.maximum(c_ref[...] + d_ref[...], 1.0))

    return pl.pallas_call(
        body,
        out_shape=(
            jax.ShapeDtypeStruct((NPAD, 1), jnp.float32),
            jax.ShapeDtypeStruct((NPAD, 1), jnp.float32),
        ),
    )(a, b, cc, d)


def _tc_matmul(x, W, s_out):
    BR = 1000

    def body(x_ref, w_ref, s_ref, o_ref):
        o_ref[...] = (
            jnp.dot(x_ref[...], w_ref[...], preferred_element_type=jnp.float32)
            * s_ref[...]
        )

    return pl.pallas_call(
        body,
        grid=(N // BR,),
        in_specs=[
            pl.BlockSpec((BR, D), lambda i: (i, 0)),
            pl.BlockSpec((D, D), lambda i: (0, 0)),
            pl.BlockSpec((BR, 1), lambda i: (i, 0)),
        ],
        out_specs=pl.BlockSpec((BR, D), lambda i: (i, 0)),
        out_shape=jax.ShapeDtypeStruct((N, D), jnp.float32),
    )(x, W, s_out)


def _sc_gather_scatter(h, src, dst, w):
    @functools.partial(
        pl.kernel,
        mesh=_mesh(),
        out_type=jax.ShapeDtypeStruct((NC, NPAD, D), jnp.float32),
        scratch_types=[
            pltpu.VMEM_SHARED((NPAD, D), jnp.float32),
            pltpu.VMEM((3 * SUB, D), jnp.float32),
            pltpu.VMEM((4 * SUB,), jnp.int32),
            pltpu.VMEM((4 * SUB,), jnp.int32),
            pltpu.VMEM((4 * SUB,), jnp.float32),
            pltpu.VMEM((SUB, D), jnp.float32),
            pltpu.SemaphoreType.DMA,
            pltpu.SemaphoreType.DMA,
            pltpu.SemaphoreType.DMA,
        ],
    )
    def k(h_h, src_h, dst_h, w_h, out_h,
          acc_sh, rows_v, sq, dq, wq, zv, ld_sem, g_sem, sc_sem):
        c = lax.axis_index("c")
        s = lax.axis_index("s")
        wid = s * NC + c
        ebase = wid * EPW

        def fillz(i, carry):
            for cc in range(8):
                zv[i, pl.ds(cc * 16, 16)] = jnp.zeros((16,), jnp.float32)
            return carry

        lax.fori_loop(0, SUB, fillz, 0)

        def fillq(i, carry):
            sq[pl.ds(i * 16, 16)] = jnp.zeros((16,), jnp.int32)
            dq[pl.ds(i * 16, 16)] = jnp.zeros((16,), jnp.int32)
            return carry

        lax.fori_loop(0, (4 * SUB) // 16, fillq, 0)

        zs = [
            pltpu.async_copy(zv, acc_sh.at[pl.ds(s * SPT + j * SUB, SUB)],
                             ld_sem)
            for j in range(SPT // SUB)
        ]
        for z in zs:
            z.wait()
        plsc.subcore_barrier()

        def q4(t):
            return lax.rem(t, 4) * SUB

        def r3(t):
            return lax.rem(t, 3) * SUB

        def issue_loads(t):
            pltpu.async_copy(src_h.at[pl.ds(ebase + t * SUB, SUB)],
                             sq.at[pl.ds(q4(t), SUB)], ld_sem)
            pltpu.async_copy(dst_h.at[pl.ds(ebase + t * SUB, SUB)],
                             dq.at[pl.ds(q4(t), SUB)], ld_sem)
            pltpu.async_copy(w_h.at[pl.ds(ebase + t * SUB, SUB)],
                             wq.at[pl.ds(q4(t), SUB)], ld_sem)

        def wait_loads(t):
            pltpu.make_async_copy(src_h.at[pl.ds(ebase + t * SUB, SUB)],
                                  sq.at[pl.ds(q4(t), SUB)], ld_sem).wait()
            pltpu.make_async_copy(dst_h.at[pl.ds(ebase + t * SUB, SUB)],
                                  dq.at[pl.ds(q4(t), SUB)], ld_sem).wait()
            pltpu.make_async_copy(w_h.at[pl.ds(ebase + t * SUB, SUB)],
                                  wq.at[pl.ds(q4(t), SUB)], ld_sem).wait()

        def issue_gather(t):
            pltpu.async_copy(h_h.at[sq.at[pl.ds(q4(t), SUB)]],
                             rows_v.at[pl.ds(r3(t), SUB)], g_sem)

        def wait_gather(t):
            pltpu.make_async_copy(h_h.at[sq.at[pl.ds(q4(t), SUB)]],
                                  rows_v.at[pl.ds(r3(t), SUB)], g_sem).wait()

        def wait_scatter(t):
            pltpu.make_async_copy(
                rows_v.at[pl.ds(r3(t), SUB)],
                acc_sh.at[dq.at[pl.ds(q4(t), SUB)]], sc_sem).wait()

        pltpu.async_copy(zv, acc_sh.at[dq.at[pl.ds(2 * SUB, SUB)]],
                         sc_sem, add=True)
        pltpu.async_copy(zv, acc_sh.at[dq.at[pl.ds(3 * SUB, SUB)]],
                         sc_sem, add=True)
        issue_loads(0)
        issue_loads(1)
        wait_loads(0)
        issue_gather(0)

        dnums = lax.GatherDimensionNumbers(
            offset_dims=(), collapsed_slice_dims=(0,), start_index_map=(0,)
        )

        NB = EPW // SUB

        def body(bi, carry):
            pltpu.make_async_copy(
                rows_v.at[pl.ds(r3(bi + 1), SUB)],
                acc_sh.at[dq.at[pl.ds(q4(bi + 2), SUB)]], sc_sem).wait()
            wait_loads(bi + 1)
            issue_gather(bi + 1)
            issue_loads(bi + 2)
            wait_gather(bi)
            rbase = r3(bi)
            qb = q4(bi)

            def mul(g, carry2):
                wchunk = wq[pl.ds(qb + g * 16, 16)]
                for lane in range(16):
                    wv = lax.gather(
                        wchunk,
                        jnp.full((16, 1), lane, jnp.int32),
                        dnums,
                        (1,),
                        mode=lax.GatherScatterMode.PROMISE_IN_BOUNDS,
                    )
                    jj = rbase + g * 16 + lane
                    for cc in range(8):
                        sl = pl.ds(cc * 16, 16)
                        rows_v[jj, sl] = rows_v[jj, sl] * wv
                return carry2

            lax.fori_loop(0, SUB // 16, mul, 0)

            pltpu.async_copy(rows_v.at[pl.ds(rbase, SUB)],
                             acc_sh.at[dq.at[pl.ds(qb, SUB)]],
                             sc_sem, add=True)
            return carry

        lax.fori_loop(0, NB, body, 0)

        for t in (NB - 2, NB - 1):
            wait_scatter(t)
        wait_gather(NB)
        wait_loads(NB + 1)
        plsc.subcore_barrier()
        for j in range(SPT // 160):
            pltpu.sync_copy(
                acc_sh.at[pl.ds(s * SPT + j * 160, 160)],
                rows_v.at[pl.ds(0, 160)],
            )
            pltpu.sync_copy(
                rows_v.at[pl.ds(0, 160)],
                out_h.at[c, pl.ds(s * SPT + j * 160, 160)],
            )

    return k(h, src, dst, w)


def _tc_combine(p, s_in, b2):
    BR = 1000

    def body(p_ref, s_ref, b_ref, o_ref):
        o_ref[...] = (p_ref[0] + p_ref[1]) * s_ref[...] + b_ref[...]

    return pl.pallas_call(
        body,
        grid=(N // BR,),
        in_specs=[
            pl.BlockSpec((2, BR, D), lambda i: (0, i, 0)),
            pl.BlockSpec((BR, 1), lambda i: (i, 0)),
            pl.BlockSpec((1, D), lambda i: (0, 0)),
        ],
        out_specs=pl.BlockSpec((BR, D), lambda i: (i, 0)),
        out_shape=jax.ShapeDtypeStruct((N, D), jnp.float32),
    )(p, s_in, b2)


def kernel(x, edge_index, edge_weight, W, b):
    pad = 2 * SUB
    src = jnp.pad(edge_index[0].astype(jnp.int32), (0, pad))
    dst = jnp.pad(edge_index[1].astype(jnp.int32), (0, pad))
    w = jnp.pad(edge_weight.astype(jnp.float32), (0, pad))
    dego_p, degi_p = _sc_degrees(src, dst)
    dego_p = dego_p.reshape(NC, NPAD)
    degi_p = degi_p.reshape(NC, NPAD)
    s_out, s_in = _tc_scales(
        dego_p[0].reshape(NPAD, 1),
        dego_p[1].reshape(NPAD, 1),
        degi_p[0].reshape(NPAD, 1),
        degi_p[1].reshape(NPAD, 1),
    )
    h = _tc_matmul(x, W, s_out[:N])
    partials = _sc_gather_scatter(h, src, dst, w)
    out = _tc_combine(partials[:, :N, :], s_in[:N], b.reshape(1, D))
    return out
